# baseline (device time: 99110 ns/iter reference)
import jax
import jax.numpy as jnp
from jax import lax
from jax.experimental import pallas as pl
from jax.experimental.pallas import tpu as pltpu

P = 8
B = 2
SQ = 256
D = 512
HD = 256
NH = 4
DH = 64
SKV0 = 256
SKV1 = 128
SKV = SKV0 + SKV1
WIN = 128
CH = SQ // P
MESH = pl.DeviceIdType.MESH
BF16 = jnp.bfloat16
QSCALE = 127.0 / 4.0
INV_QS = 4.0 / 127.0

SEND_ORDERS = {
    0: (1, 2, 5, 6, 3, 7, 4),
    1: (0, 3, 4, 7, 2, 6, 5),
}


def kernel(x, Wq, K_ext, V_ext, Wo):
    Kf = K_ext
    Vf = V_ext

    def body(x_ref, wq_ref, kf_hbm, vf_hbm, wo_ref, out_ref,
             kbuf, vbuf, ldk, ldv, kq, vq, part, rs_buf, ag_buf,
             ld_sems, kv_send_sems, kv_recv_sems,
             rs_send_sems, rs_recv_sems, ag_send_sems, ag_recv_sems):
        my = lax.axis_index("i")

        bar = pltpu.get_barrier_semaphore()
        for k in range(1, P):
            tgt = lax.rem(my + k, P)
            pl.semaphore_signal(bar, inc=1, device_id=(tgt,),
                                device_id_type=MESH)

        def quant_flat(ld, slot, rows, write):
            for h in range(NH):
                write(h, jnp.clip(
                    jnp.round(ld[slot, :, 0:rows, h, :] * QSCALE),
                    -127, 127).astype(jnp.int8))

        def sender(src_id, rows, dst_lo, dst_hi):
            order = SEND_ORDERS[src_id]
            n = len(order)

            def make_load(i, d):
                for t, (hbm, ld) in enumerate(((kf_hbm, ldk), (vf_hbm, ldv))):
                    yield pltpu.make_async_copy(
                        hbm.at[:, 0:rows, NH * d:NH * (d + 1), :],
                        ld.at[i % 2, :, 0:rows, :, :],
                        ld_sems.at[i % 2, t],
                    )

            def start_load(i):
                for cp in make_load(i, order[i]):
                    cp.start()

            start_load(0)
            start_load(1)
            barrier_done = False
            for i, d in enumerate(order):
                for cp in make_load(i, d):
                    cp.wait()
                for stage, ld in ((kq, ldk), (vq, ldv)):
                    quant_flat(ld, i % 2, rows, lambda h, v, stage=stage: (
                        stage.__setitem__(
                            (d, slice(None), slice(0, rows),
                             slice(h * DH, (h + 1) * DH)), v)))
                if i + 2 < n:
                    start_load(i + 2)
                if not barrier_done:
                    pl.semaphore_wait(bar, P - 1)
                    barrier_done = True
                for t, (stage, buf) in enumerate(((kq, kbuf), (vq, vbuf))):
                    pltpu.make_async_remote_copy(
                        src_ref=stage.at[d, :, 0:rows, :],
                        dst_ref=buf.at[:, dst_lo:dst_hi, :],
                        send_sem=kv_send_sems.at[d, t],
                        recv_sem=kv_recv_sems.at[src_id, t],
                        device_id=(d,), device_id_type=MESH,
                    ).start()
            slot = n % 2
            for cp in make_load(n, src_id):
                cp.start()
            for cp in make_load(n, src_id):
                cp.wait()
            for buf, ld in ((kbuf, ldk), (vbuf, ldv)):
                quant_flat(ld, slot, rows, lambda h, v, buf=buf: (
                    buf.__setitem__(
                        (slice(None), slice(dst_lo, dst_hi),
                         slice(h * DH, (h + 1) * DH)), v)))

        @pl.when(my == 0)
        def _():
            sender(0, SKV0, 0, SKV0)

        @pl.when(my == 1)
        def _():
            sender(1, SKV1, SKV0, SKV)

        @pl.when(jnp.logical_and(my != 0, my != 1))
        def _():
            pl.semaphore_wait(bar, P - 1)

        q = [jnp.dot(x_ref[b], wq_ref[:, :],
                     preferred_element_type=jnp.float32).astype(BF16)
             for b in range(B)]

        @pl.when(my != 0)
        def _():
            for t, buf in enumerate((kbuf, vbuf)):
                pltpu.make_async_remote_copy(
                    src_ref=kq.at[0, :, 0:SKV0, :],
                    dst_ref=buf.at[:, 0:SKV0, :],
                    send_sem=kv_send_sems.at[0, t],
                    recv_sem=kv_recv_sems.at[0, t],
                    device_id=(0,), device_id_type=MESH,
                ).wait_recv()

        @pl.when(my != 1)
        def _():
            for t, buf in enumerate((kbuf, vbuf)):
                pltpu.make_async_remote_copy(
                    src_ref=kq.at[0, :, 0:SKV1, :],
                    dst_ref=buf.at[:, SKV0:SKV, :],
                    send_sem=kv_send_sems.at[1, t],
                    recv_sem=kv_recv_sems.at[1, t],
                    device_id=(0,), device_id_type=MESH,
                ).wait_recv()

        qi = lax.broadcasted_iota(jnp.int32, (SQ, SKV), 0)
        ki = lax.broadcasted_iota(jnp.int32, (SQ, SKV), 1)
        mask = jnp.abs(qi - ki) <= WIN

        for b in range(B):
            ctxs = []
            for h in range(NH):
                q_bh = q[b][:, h * DH:(h + 1) * DH]
                k_bh = kbuf[b, :, h * DH:(h + 1) * DH].astype(BF16)
                v_bh = vbuf[b, :, h * DH:(h + 1) * DH].astype(BF16)
                s = lax.dot_general(
                    q_bh, k_bh, (((1,), (1,)), ((), ())),
                    preferred_element_type=jnp.float32) * (0.125 * INV_QS)
                s = jnp.where(mask, s, -1e9)
                m = jnp.max(s, axis=-1, keepdims=True)
                w = jnp.exp(s - m)
                w = (w * (INV_QS / jnp.sum(w, axis=-1, keepdims=True))
                     ).astype(BF16)
                ctxs.append(jnp.dot(w, v_bh,
                                    preferred_element_type=jnp.float32))
            ctx_b = jnp.concatenate(ctxs, axis=1)
            part[b] = jnp.dot(ctx_b, wo_ref[:, :],
                              preferred_element_type=jnp.float32).astype(BF16)

        for d in range(P):
            @pl.when(my != d)
            def _(d=d):
                pltpu.make_async_remote_copy(
                    src_ref=part.at[:, d * CH:(d + 1) * CH, :],
                    dst_ref=rs_buf.at[my],
                    send_sem=rs_send_sems.at[d],
                    recv_sem=rs_recv_sems.at[my],
                    device_id=(d,), device_id_type=MESH,
                ).start()
        rs_buf[my] = part[:, pl.ds(my * CH, CH), :]

        for j in range(P):
            @pl.when(my != j)
            def _(j=j):
                pltpu.make_async_remote_copy(
                    src_ref=part.at[:, 0:CH, :],
                    dst_ref=rs_buf.at[j],
                    send_sem=rs_send_sems.at[j],
                    recv_sem=rs_recv_sems.at[j],
                    device_id=(0,), device_id_type=MESH,
                ).wait_recv()

        red = rs_buf[0].astype(jnp.float32)
        for j in range(1, P):
            red = red + rs_buf[j].astype(jnp.float32)
        out_ref[:, pl.ds(my * CH, CH), :] = red
        ag_buf[my] = red.astype(BF16)

        for d in range(P):
            @pl.when(my != d)
            def _(d=d):
                pltpu.make_async_remote_copy(
                    src_ref=ag_buf.at[my],
                    dst_ref=ag_buf.at[my],
                    send_sem=ag_send_sems.at[d],
                    recv_sem=ag_recv_sems.at[my],
                    device_id=(d,), device_id_type=MESH,
                ).start()

        for j in range(P):
            @pl.when(my != j)
            def _(j=j):
                pltpu.make_async_remote_copy(
                    src_ref=ag_buf.at[0],
                    dst_ref=ag_buf.at[j],
                    send_sem=ag_send_sems.at[j],
                    recv_sem=ag_recv_sems.at[j],
                    device_id=(0,), device_id_type=MESH,
                ).wait_recv()
                out_ref[:, j * CH:(j + 1) * CH, :] = (
                    ag_buf[j].astype(jnp.float32))

        def drain_kv(src_id, rows, dst_lo, dst_hi):
            for d in SEND_ORDERS[src_id]:
                for t, (stage, buf) in enumerate(((kq, kbuf), (vq, vbuf))):
                    pltpu.make_async_remote_copy(
                        src_ref=stage.at[d, :, 0:rows, :],
                        dst_ref=buf.at[:, dst_lo:dst_hi, :],
                        send_sem=kv_send_sems.at[d, t],
                        recv_sem=kv_recv_sems.at[src_id, t],
                        device_id=(d,), device_id_type=MESH,
                    ).wait_send()

        @pl.when(my == 0)
        def _():
            drain_kv(0, SKV0, 0, SKV0)

        @pl.when(my == 1)
        def _():
            drain_kv(1, SKV1, SKV0, SKV)

        for d in range(P):
            @pl.when(my != d)
            def _(d=d):
                pltpu.make_async_remote_copy(
                    src_ref=part.at[:, d * CH:(d + 1) * CH, :],
                    dst_ref=rs_buf.at[my],
                    send_sem=rs_send_sems.at[d],
                    recv_sem=rs_recv_sems.at[my],
                    device_id=(d,), device_id_type=MESH,
                ).wait_send()
                pltpu.make_async_remote_copy(
                    src_ref=ag_buf.at[my],
                    dst_ref=ag_buf.at[my],
                    send_sem=ag_send_sems.at[d],
                    recv_sem=ag_recv_sems.at[my],
                    device_id=(d,), device_id_type=MESH,
                ).wait_send()

    return pl.pallas_call(
        body,
        out_shape=jax.ShapeDtypeStruct((B, SQ, D), jnp.float32),
        in_specs=[
            pl.BlockSpec(memory_space=pltpu.VMEM),
            pl.BlockSpec(memory_space=pltpu.VMEM),
            pl.BlockSpec(memory_space=pl.ANY),
            pl.BlockSpec(memory_space=pl.ANY),
            pl.BlockSpec(memory_space=pltpu.VMEM),
        ],
        out_specs=pl.BlockSpec(memory_space=pltpu.VMEM),
        scratch_shapes=[
            pltpu.VMEM((B, SKV, HD), jnp.int8),
            pltpu.VMEM((B, SKV, HD), jnp.int8),
            pltpu.VMEM((2, B, SKV0, NH, DH), jnp.float32),
            pltpu.VMEM((2, B, SKV0, NH, DH), jnp.float32),
            pltpu.VMEM((P, B, SKV0, HD), jnp.int8),
            pltpu.VMEM((P, B, SKV0, HD), jnp.int8),
            pltpu.VMEM((B, SQ, D), BF16),
            pltpu.VMEM((P, B, CH, D), BF16),
            pltpu.VMEM((P, B, CH, D), BF16),
            pltpu.SemaphoreType.DMA((2, 2)),
            pltpu.SemaphoreType.DMA((P, 2)),
            pltpu.SemaphoreType.DMA((2, 2)),
            pltpu.SemaphoreType.DMA((P,)),
            pltpu.SemaphoreType.DMA((P,)),
            pltpu.SemaphoreType.DMA((P,)),
            pltpu.SemaphoreType.DMA((P,)),
        ],
        compiler_params=pltpu.CompilerParams(collective_id=0),
    )(x, Wq, Kf, Vf, Wo)


# device time: 38724 ns/iter; 2.5594x vs baseline; 2.5594x over previous
import jax
import jax.numpy as jnp
from jax import lax
from jax.experimental import pallas as pl
from jax.experimental.pallas import tpu as pltpu

P = 8
B = 2
SQ = 256
D = 512
HD = 256
NH = 4
DH = 64
SKV0 = 256
SKV1 = 128
SKV = SKV0 + SKV1
WIN = 128
CH = SQ // P
MESH = pl.DeviceIdType.MESH
BF16 = jnp.bfloat16
QSCALE = 127.0 / 4.0
INV_QS = 4.0 / 127.0


def kernel(x, Wq, K_ext, V_ext, Wo):
    def q8(a):
        a = a.reshape(B, SKV0, P * HD)
        return jnp.clip(jnp.round(a * QSCALE), -127, 127).astype(jnp.int8)

    Kf = q8(K_ext)
    Vf = q8(V_ext)

    def body(x_ref, wq_ref, kf_ref, vf_ref, wo_ref, out_ref,
             kbuf, vbuf, part, rs_buf, ag_buf,
             kv_send_sems, kv_recv_sems,
             rs_send_sems, rs_recv_sems, ag_send_sems, ag_recv_sems):
        my = lax.axis_index("i")

        bar = pltpu.get_barrier_semaphore()
        for k in range(1, P):
            tgt = lax.rem(my + k, P)
            pl.semaphore_signal(bar, inc=1, device_id=(tgt,),
                                device_id_type=MESH)
        pl.semaphore_wait(bar, P - 1)

        @pl.when(my == 0)
        def _():
            for d in range(1, P):
                for t, (src, buf) in enumerate(((kf_ref, kbuf), (vf_ref, vbuf))):
                    pltpu.make_async_remote_copy(
                        src_ref=src.at[:, :, d * HD:(d + 1) * HD],
                        dst_ref=buf.at[:, 0:SKV0, :],
                        send_sem=kv_send_sems.at[d, t],
                        recv_sem=kv_recv_sems.at[0, t],
                        device_id=(d,), device_id_type=MESH,
                    ).start()
            kbuf[:, 0:SKV0, :] = kf_ref[:, :, 0:HD]
            vbuf[:, 0:SKV0, :] = vf_ref[:, :, 0:HD]

        @pl.when(my == 1)
        def _():
            for d in [0] + list(range(2, P)):
                for t, (src, buf) in enumerate(((kf_ref, kbuf), (vf_ref, vbuf))):
                    pltpu.make_async_remote_copy(
                        src_ref=src.at[:, 0:SKV1, d * HD:(d + 1) * HD],
                        dst_ref=buf.at[:, SKV0:SKV, :],
                        send_sem=kv_send_sems.at[d, t],
                        recv_sem=kv_recv_sems.at[1, t],
                        device_id=(d,), device_id_type=MESH,
                    ).start()
            kbuf[:, SKV0:SKV, :] = kf_ref[:, 0:SKV1, HD:2 * HD]
            vbuf[:, SKV0:SKV, :] = vf_ref[:, 0:SKV1, HD:2 * HD]

        q = [jnp.dot(x_ref[b], wq_ref[:, :],
                     preferred_element_type=jnp.float32).astype(BF16)
             for b in range(B)]

        def wait_shard0():
            @pl.when(my != 0)
            def _():
                for t, buf in enumerate((kbuf, vbuf)):
                    pltpu.make_async_remote_copy(
                        src_ref=kf_ref.at[:, :, 0:HD],
                        dst_ref=buf.at[:, 0:SKV0, :],
                        send_sem=kv_send_sems.at[0, t],
                        recv_sem=kv_recv_sems.at[0, t],
                        device_id=(0,), device_id_type=MESH,
                    ).wait_recv()

        def wait_shard1():
            @pl.when(my != 1)
            def _():
                for t, buf in enumerate((kbuf, vbuf)):
                    pltpu.make_async_remote_copy(
                        src_ref=kf_ref.at[:, 0:SKV1, 0:HD],
                        dst_ref=buf.at[:, SKV0:SKV, :],
                        send_sem=kv_send_sems.at[1, t],
                        recv_sem=kv_recv_sems.at[1, t],
                        device_id=(0,), device_id_type=MESH,
                    ).wait_recv()

        HQ = SQ // 2

        def attn_half(half, kv_hi):
            r0 = half * HQ
            qi = lax.broadcasted_iota(jnp.int32, (HQ, kv_hi), 0) + r0
            ki = lax.broadcasted_iota(jnp.int32, (HQ, kv_hi), 1)
            mask = jnp.abs(qi - ki) <= WIN
            for b in range(B):
                ctxs = []
                for h in range(NH):
                    q_bh = q[b][r0:r0 + HQ, h * DH:(h + 1) * DH]
                    k_bh = kbuf[b, 0:kv_hi, h * DH:(h + 1) * DH].astype(BF16)
                    v_bh = vbuf[b, 0:kv_hi, h * DH:(h + 1) * DH].astype(BF16)
                    s = lax.dot_general(
                        q_bh, k_bh, (((1,), (1,)), ((), ())),
                        preferred_element_type=jnp.float32) * (0.125 * INV_QS)
                    s = jnp.where(mask, s, -1e9)
                    m = jnp.max(s, axis=-1, keepdims=True)
                    w = jnp.exp(s - m)
                    w = (w * (INV_QS / jnp.sum(w, axis=-1, keepdims=True))
                         ).astype(BF16)
                    ctxs.append(jnp.dot(w, v_bh,
                                        preferred_element_type=jnp.float32))
                ctx_b = jnp.concatenate(ctxs, axis=1)
                part[b, r0:r0 + HQ, :] = jnp.dot(
                    ctx_b, wo_ref[:, :],
                    preferred_element_type=jnp.float32).astype(BF16)

        def rs_quarter(half):
            for d in range(4 * half, 4 * half + 4):
                @pl.when(my != d)
                def _(d=d):
                    pltpu.make_async_remote_copy(
                        src_ref=part.at[:, d * CH:(d + 1) * CH, :],
                        dst_ref=rs_buf.at[my],
                        send_sem=rs_send_sems.at[d],
                        recv_sem=rs_recv_sems.at[my],
                        device_id=(d,), device_id_type=MESH,
                    ).start()
            in_half = (my < 4) if half == 0 else (my >= 4)
            @pl.when(in_half)
            def _():
                rs_buf[my] = part[:, pl.ds(my * CH, CH), :]

        wait_shard0()
        attn_half(0, SKV0)
        rs_quarter(0)
        wait_shard1()
        attn_half(1, SKV)
        rs_quarter(1)

        def reduce_and_ag(active):
            @pl.when(active)
            def _():
                for j in range(P):
                    @pl.when(my != j)
                    def _(j=j):
                        pltpu.make_async_remote_copy(
                            src_ref=part.at[:, 0:CH, :],
                            dst_ref=rs_buf.at[j],
                            send_sem=rs_send_sems.at[j],
                            recv_sem=rs_recv_sems.at[j],
                            device_id=(0,), device_id_type=MESH,
                        ).wait_recv()
                red = rs_buf[0].astype(jnp.float32)
                for j in range(1, P):
                    red = red + rs_buf[j].astype(jnp.float32)
                out_ref[:, pl.ds(my * CH, CH), :] = red
                ag_buf[my] = red.astype(BF16)
                for d in range(P):
                    @pl.when(my != d)
                    def _(d=d):
                        pltpu.make_async_remote_copy(
                            src_ref=ag_buf.at[my],
                            dst_ref=ag_buf.at[my],
                            send_sem=ag_send_sems.at[d],
                            recv_sem=ag_recv_sems.at[my],
                            device_id=(d,), device_id_type=MESH,
                        ).start()

        reduce_and_ag(my < 4)
        reduce_and_ag(my >= 4)

        for j in range(P):
            @pl.when(my != j)
            def _(j=j):
                pltpu.make_async_remote_copy(
                    src_ref=ag_buf.at[0],
                    dst_ref=ag_buf.at[j],
                    send_sem=ag_send_sems.at[j],
                    recv_sem=ag_recv_sems.at[j],
                    device_id=(0,), device_id_type=MESH,
                ).wait_recv()
                out_ref[:, j * CH:(j + 1) * CH, :] = (
                    ag_buf[j].astype(jnp.float32))

        @pl.when(my == 0)
        def _():
            for d in range(1, P):
                for t, (src, buf) in enumerate(((kf_ref, kbuf), (vf_ref, vbuf))):
                    pltpu.make_async_remote_copy(
                        src_ref=src.at[:, :, d * HD:(d + 1) * HD],
                        dst_ref=buf.at[:, 0:SKV0, :],
                        send_sem=kv_send_sems.at[d, t],
                        recv_sem=kv_recv_sems.at[0, t],
                        device_id=(d,), device_id_type=MESH,
                    ).wait_send()

        @pl.when(my == 1)
        def _():
            for d in [0] + list(range(2, P)):
                for t, (src, buf) in enumerate(((kf_ref, kbuf), (vf_ref, vbuf))):
                    pltpu.make_async_remote_copy(
                        src_ref=src.at[:, 0:SKV1, d * HD:(d + 1) * HD],
                        dst_ref=buf.at[:, SKV0:SKV, :],
                        send_sem=kv_send_sems.at[d, t],
                        recv_sem=kv_recv_sems.at[1, t],
                        device_id=(d,), device_id_type=MESH,
                    ).wait_send()

        for d in range(P):
            @pl.when(my != d)
            def _(d=d):
                pltpu.make_async_remote_copy(
                    src_ref=part.at[:, d * CH:(d + 1) * CH, :],
                    dst_ref=rs_buf.at[my],
                    send_sem=rs_send_sems.at[d],
                    recv_sem=rs_recv_sems.at[my],
                    device_id=(d,), device_id_type=MESH,
                ).wait_send()
                pltpu.make_async_remote_copy(
                    src_ref=ag_buf.at[my],
                    dst_ref=ag_buf.at[my],
                    send_sem=ag_send_sems.at[d],
                    recv_sem=ag_recv_sems.at[my],
                    device_id=(d,), device_id_type=MESH,
                ).wait_send()

    return pl.pallas_call(
        body,
        out_shape=jax.ShapeDtypeStruct((B, SQ, D), jnp.float32),
        in_specs=[pl.BlockSpec(memory_space=pltpu.VMEM)] * 5,
        out_specs=pl.BlockSpec(memory_space=pltpu.VMEM),
        scratch_shapes=[
            pltpu.VMEM((B, SKV, HD), jnp.int8),
            pltpu.VMEM((B, SKV, HD), jnp.int8),
            pltpu.VMEM((B, SQ, D), BF16),
            pltpu.VMEM((P, B, CH, D), BF16),
            pltpu.VMEM((P, B, CH, D), BF16),
            pltpu.SemaphoreType.DMA((P, 2)),
            pltpu.SemaphoreType.DMA((2, 2)),
            pltpu.SemaphoreType.DMA((P,)),
            pltpu.SemaphoreType.DMA((P,)),
            pltpu.SemaphoreType.DMA((P,)),
            pltpu.SemaphoreType.DMA((P,)),
        ],
        compiler_params=pltpu.CompilerParams(collective_id=0),
    )(x, Wq, Kf, Vf, Wo)
